# baseline (speedup 1.0000x reference)
import functools

import jax
import jax.numpy as jnp
from jax.experimental import pallas as pl
from jax.experimental.pallas import tpu as pltpu

_B, _D, _C, _K, _L = 64, 2048, 345, 40, 9


def _head_body(f_ref, W_ref, b_ref, lab_ref, ptr_ref,
               logits_ref, z_ref, tgt_ref, nptr_ref):
    f = f_ref[...]
    W = W_ref[...]
    logits = jnp.dot(f, W, preferred_element_type=jnp.float32) + b_ref[...]
    logits_ref[...] = logits
    norm = jnp.sqrt(jnp.sum(logits * logits, axis=1, keepdims=True))
    z_ref[...] = logits / (norm + 1e-12)

    lab = lab_ref[...]
    ptr = ptr_ref[...]
    lab_i = lab.reshape(_B, 1)
    same = lab_i == lab
    rows = jax.lax.broadcasted_iota(jnp.int32, (_B, _B), 0)
    cols = jax.lax.broadcasted_iota(jnp.int32, (_B, _B), 1)
    occ = jnp.sum(jnp.where(same & (cols < rows), 1, 0), axis=1,
                  dtype=jnp.int32).reshape(1, _B)
    cids = jax.lax.broadcasted_iota(jnp.int32, (_B, _C), 1)
    onehot_lab = lab_i == cids
    ptr_g = jnp.sum(jnp.where(onehot_lab, ptr, 0), axis=1,
                    dtype=jnp.int32).reshape(1, _B)
    pos = jax.lax.rem(ptr_g + occ, _K)
    tgt_ref[...] = lab * _K + pos
    counts = jnp.sum(jnp.where(onehot_lab, 1, 0), axis=0,
                     dtype=jnp.int32).reshape(1, _C)
    nptr_ref[...] = jax.lax.rem(ptr + counts, _K)


def _qz_body(qz_ref, z_ref, tgt_ref, out_ref, *, lpb):
    j = pl.program_id(0)
    tgt = tgt_ref[...].reshape(_B, 1)
    slots = jax.lax.broadcasted_iota(jnp.int32, (_B, lpb), 1) + j * lpb
    onehot = (tgt == slots).astype(jnp.float32)
    val = jax.lax.dot_general(
        z_ref[...], onehot, (((0,), (0,)), ((), ())),
        preferred_element_type=jnp.float32)
    written = jnp.max(onehot, axis=0, keepdims=True) > 0.5
    out_ref[...] = jnp.where(written, val, qz_ref[...])


def _qh_body(qh_ref, h_ref, tgt_ref, out_ref, *, rb):
    j = pl.program_id(0)
    tgt = tgt_ref[...].reshape(_B, 1)
    rows = jax.lax.broadcasted_iota(jnp.int32, (_B, rb), 1) + j * rb
    onehot = (tgt == rows).astype(jnp.float32)
    val = jax.lax.dot_general(
        onehot, h_ref[...], (((0,), (0,)), ((), ())),
        preferred_element_type=jnp.float32)
    written = jnp.max(onehot, axis=0).reshape(rb, 1) > 0.5
    out_ref[...] = jnp.where(written, val, qh_ref[...])


def kernel(f, labels, h, queue_z, queue_h, queue_ptr, W, b):
    B, D, C, K, L = _B, _D, _C, _K, _L
    lab2 = labels.reshape(1, B)
    ptr2 = queue_ptr.reshape(1, C)
    b2 = b.reshape(1, C)

    logits, z, tgt, nptr = pl.pallas_call(
        _head_body,
        out_shape=(
            jax.ShapeDtypeStruct((B, C), jnp.float32),
            jax.ShapeDtypeStruct((B, C), jnp.float32),
            jax.ShapeDtypeStruct((1, B), jnp.int32),
            jax.ShapeDtypeStruct((1, C), jnp.int32),
        ),
    )(f, W, b2, lab2, ptr2)

    LPB = 1280
    qz2 = queue_z.reshape(C, C * K)
    new_qz = pl.pallas_call(
        functools.partial(_qz_body, lpb=LPB),
        grid=(pl.cdiv(C * K, LPB),),
        in_specs=[
            pl.BlockSpec((C, LPB), lambda i: (0, i)),
            pl.BlockSpec((B, C), lambda i: (0, 0)),
            pl.BlockSpec((1, B), lambda i: (0, 0)),
        ],
        out_specs=pl.BlockSpec((C, LPB), lambda i: (0, i)),
        out_shape=jax.ShapeDtypeStruct((C, C * K), jnp.float32),
    )(qz2, z, tgt).reshape(C, C, K)

    RB = 200
    qh2 = queue_h.reshape(C * K, L * C)
    h2 = h.reshape(B, L * C)
    new_qh = pl.pallas_call(
        functools.partial(_qh_body, rb=RB),
        grid=(C * K // RB,),
        in_specs=[
            pl.BlockSpec((RB, L * C), lambda i: (i, 0)),
            pl.BlockSpec((B, L * C), lambda i: (0, 0)),
            pl.BlockSpec((1, B), lambda i: (0, 0)),
        ],
        out_specs=pl.BlockSpec((RB, L * C), lambda i: (i, 0)),
        out_shape=jax.ShapeDtypeStruct((C * K, L * C), jnp.float32),
    )(qh2, h2, tgt).reshape(C, K, L, C)

    return (logits, new_qz, new_qh, nptr.reshape(C))
